# baseline (speedup 1.0000x reference)
import functools

import jax
import jax.numpy as jnp
from jax import lax
from jax.experimental import pallas as pl
from jax.experimental.pallas import tpu as pltpu
from jax.experimental.pallas import tpu_sc as plsc

_ALPHA = 16.0
_NEG = -1e30


def _pass_a_body(x_ref, aaug_ref, h_ref, ls_ref, *, e, r):
    si = pl.program_id(1)
    x = x_ref[0]
    hf = jax.lax.dot_general(
        x.astype(jnp.bfloat16), aaug_ref[...], (((1,), (1,)), ((), ())),
        preferred_element_type=jnp.float32)
    h_ref[0] = hf.astype(jnp.bfloat16)
    part = jnp.sum(hf[:, e * r:e * r + 2 * e], axis=0, keepdims=True)

    @pl.when(si == 0)
    def _():
        ls_ref[0] = part

    @pl.when(si != 0)
    def _():
        ls_ref[0] = ls_ref[0] + part


def _take(x, idx):
    return x.at[idx].get(mode="promise_in_bounds")


def _bfly_max(x):
    iota = lax.broadcasted_iota(jnp.int32, (16,), 0)
    for k in (8, 4, 2, 1):
        x = jnp.maximum(x, _take(x, iota ^ k))
    return x


def _bfly_min(x):
    iota = lax.broadcasted_iota(jnp.int32, (16,), 0)
    for k in (8, 4, 2, 1):
        x = jnp.minimum(x, _take(x, iota ^ k))
    return x


def _sc_router_body(ls_ref, bias_ref, w_ref, ls_v, bias_v, w_v,
                    *, nb, e, r, s_total):
    wid = lax.axis_index("s") * 2 + lax.axis_index("c")

    @pl.when(wid == 0)
    def _():
        pltpu.sync_copy(ls_ref, ls_v)
        pltpu.sync_copy(bias_ref, bias_v)
        iota = lax.broadcasted_iota(jnp.int32, (16,), 0)
        scaling = _ALPHA / r
        for b in range(nb):
            logits = ls_v[b] * (1.0 / s_total) + bias_v[b]
            v0 = _bfly_max(logits)
            i0 = _bfly_min(jnp.where(logits == v0, iota, 16))
            masked = jnp.where(iota == i0, _NEG, logits)
            v1 = _bfly_max(masked)
            i1 = _bfly_min(jnp.where(masked == v1, iota, 16))
            t = jnp.exp(v1 - v0)
            wv = (jnp.where(iota == i0, scaling / (1.0 + t), 0.0)
                  + jnp.where(iota == i1, scaling * t / (1.0 + t), 0.0))
            w_v[b] = wv
        pltpu.sync_copy(w_v, w_ref)


def _pass_b_body(h_ref, w_ref, baug_ref, out_ref, bs_ref):
    si = pl.program_id(1)

    @pl.when(si == 0)
    def _():
        rowexp = lax.broadcasted_iota(jnp.int32, (256, 16), 0) // 16
        lane = lax.broadcasted_iota(jnp.int32, (256, 16), 1)
        wcol = jnp.sum(jnp.where(rowexp == lane, w_ref[0], 0.0),
                       axis=1, keepdims=True)
        bs_ref[...] = (baug_ref[...] * wcol).astype(jnp.bfloat16)

    out_ref[0] = jax.lax.dot_general(
        h_ref[0], bs_ref[...], (((1,), (0,)), ((), ())),
        preferred_element_type=jnp.float32)


def kernel(hidden_states, task_id, mode_id, W_gate, task_bias, mode_bias, A, Bw):
    b, s_total, d = hidden_states.shape
    e, r, _ = A.shape
    ts = 1024
    nt = s_total // ts
    kcols = 256

    a_aug = jnp.concatenate(
        [A.reshape(e * r, d).astype(jnp.bfloat16),
         W_gate.astype(jnp.bfloat16),
         jnp.zeros((kcols - e * r - e, d), jnp.bfloat16)], axis=0)
    b_aug = jnp.concatenate(
        [Bw.transpose(0, 2, 1).reshape(e * r, d),
         jnp.zeros((kcols - e * r, d), jnp.float32)], axis=0)
    bias = (jnp.take(task_bias, task_id, axis=0)
            + jnp.take(mode_bias, mode_id, axis=0))
    bias16 = jnp.concatenate(
        [bias, jnp.full((b, 2 * e - e), _NEG, jnp.float32)], axis=1)

    h, ls = pl.pallas_call(
        functools.partial(_pass_a_body, e=e, r=r),
        grid=(b, nt),
        in_specs=[
            pl.BlockSpec((1, ts, d), lambda bi, si: (bi, si, 0)),
            pl.BlockSpec((kcols, d), lambda bi, si: (0, 0)),
        ],
        out_specs=[
            pl.BlockSpec((1, ts, kcols), lambda bi, si: (bi, si, 0)),
            pl.BlockSpec((1, 1, 2 * e), lambda bi, si: (bi, 0, 0)),
        ],
        out_shape=[
            jax.ShapeDtypeStruct((b, s_total, kcols), jnp.bfloat16),
            jax.ShapeDtypeStruct((b, 1, 2 * e), jnp.float32),
        ],
    )(hidden_states, a_aug)

    mesh = plsc.VectorSubcoreMesh(core_axis_name="c", subcore_axis_name="s")
    w_exp = pl.kernel(
        functools.partial(_sc_router_body, nb=b, e=e, r=r, s_total=s_total),
        mesh=mesh,
        out_type=jax.ShapeDtypeStruct((b, 2 * e), jnp.float32),
        scratch_types=[
            pltpu.VMEM((b, 2 * e), jnp.float32),
            pltpu.VMEM((b, 2 * e), jnp.float32),
            pltpu.VMEM((b, 2 * e), jnp.float32),
        ],
    )(ls.reshape(b, 2 * e), bias16)

    return pl.pallas_call(
        _pass_b_body,
        grid=(b, nt),
        in_specs=[
            pl.BlockSpec((1, ts, kcols), lambda bi, si: (bi, si, 0)),
            pl.BlockSpec((1, 1, 2 * e), lambda bi, si: (bi, 0, 0)),
            pl.BlockSpec((kcols, d), lambda bi, si: (0, 0)),
        ],
        out_specs=pl.BlockSpec((1, ts, d), lambda bi, si: (bi, si, 0)),
        out_shape=jax.ShapeDtypeStruct((b, s_total, d), jnp.float32),
        scratch_shapes=[pltpu.VMEM((kcols, d), jnp.bfloat16)],
    )(h, w_exp.reshape(b, 1, 2 * e), b_aug)


# speedup vs baseline: 1.2497x; 1.2497x over previous
import functools

import jax
import jax.numpy as jnp
from jax import lax
from jax.experimental import pallas as pl
from jax.experimental.pallas import tpu as pltpu

_ALPHA = 16.0
_NEG = -1e30


def _pass_a_body(x_ref, bias_ref, aaug_ref, h_ref, ls_ref, *, e, r, s_total):
    si = pl.program_id(1)
    x = x_ref[0]
    hf = jax.lax.dot_general(
        x.astype(jnp.bfloat16), aaug_ref[...], (((1,), (1,)), ((), ())),
        preferred_element_type=jnp.float32)
    h_ref[0] = hf.astype(jnp.bfloat16)
    part = jnp.sum(hf[:, e * r:e * r + 2 * e], axis=0, keepdims=True)

    @pl.when(si == 0)
    def _():
        ls_ref[0] = part + bias_ref[0] * float(s_total)

    @pl.when(si != 0)
    def _():
        ls_ref[0] = ls_ref[0] + part


def _pass_b_body(h_ref, ls_ref, baug_ref, out_ref, bs_ref, *, e, r, s_total):
    si = pl.program_id(1)

    @pl.when(si == 0)
    def _router():
        logits = ls_ref[0] * (1.0 / s_total)
        iota = lax.broadcasted_iota(jnp.int32, (1, 2 * e), 1)
        v0 = jnp.max(logits, keepdims=True)
        i0 = jnp.min(jnp.where(logits == v0, iota, 2 * e), keepdims=True)
        masked = jnp.where(iota == i0, _NEG, logits)
        v1 = jnp.max(masked, keepdims=True)
        i1 = jnp.min(jnp.where(masked == v1, iota, 2 * e), keepdims=True)
        t = jnp.exp(v1 - v0)
        scaling = _ALPHA / r
        wrow = (jnp.where(iota == i0, scaling / (1.0 + t), 0.0)
                + jnp.where(iota == i1, scaling * t / (1.0 + t), 0.0))
        rowexp = lax.broadcasted_iota(jnp.int32, (16 * r, 2 * e), 0) // r
        lane = lax.broadcasted_iota(jnp.int32, (16 * r, 2 * e), 1)
        wcol = jnp.sum(jnp.where(rowexp == lane, wrow, 0.0),
                       axis=1, keepdims=True)
        bs_ref[...] = (baug_ref[...] * wcol).astype(jnp.bfloat16)

    out_ref[0] = jax.lax.dot_general(
        h_ref[0], bs_ref[...], (((1,), (0,)), ((), ())),
        preferred_element_type=jnp.float32)


def kernel(hidden_states, task_id, mode_id, W_gate, task_bias, mode_bias, A, Bw):
    b, s_total, d = hidden_states.shape
    e, r, _ = A.shape
    ts = 1024
    nt = s_total // ts
    kcols = 256

    a_aug = jnp.concatenate(
        [A.reshape(e * r, d).astype(jnp.bfloat16),
         W_gate.astype(jnp.bfloat16),
         jnp.zeros((kcols - e * r - e, d), jnp.bfloat16)], axis=0)
    b_aug = jnp.concatenate(
        [Bw.transpose(0, 2, 1).reshape(e * r, d),
         jnp.zeros((kcols - e * r, d), jnp.float32)], axis=0)
    bias = (jnp.take(task_bias, task_id, axis=0)
            + jnp.take(mode_bias, mode_id, axis=0))
    bias16 = jnp.concatenate(
        [bias, jnp.full((b, e), _NEG, jnp.float32)], axis=1)

    h, ls = pl.pallas_call(
        functools.partial(_pass_a_body, e=e, r=r, s_total=s_total),
        grid=(b, nt),
        in_specs=[
            pl.BlockSpec((1, ts, d), lambda bi, si: (bi, si, 0)),
            pl.BlockSpec((1, 1, 2 * e), lambda bi, si: (bi, 0, 0)),
            pl.BlockSpec((kcols, d), lambda bi, si: (0, 0)),
        ],
        out_specs=[
            pl.BlockSpec((1, ts, kcols), lambda bi, si: (bi, si, 0)),
            pl.BlockSpec((1, 1, 2 * e), lambda bi, si: (bi, 0, 0)),
        ],
        out_shape=[
            jax.ShapeDtypeStruct((b, s_total, kcols), jnp.bfloat16),
            jax.ShapeDtypeStruct((b, 1, 2 * e), jnp.float32),
        ],
    )(hidden_states, bias16.reshape(b, 1, 2 * e), a_aug)

    return pl.pallas_call(
        functools.partial(_pass_b_body, e=e, r=r, s_total=s_total),
        grid=(b, nt),
        in_specs=[
            pl.BlockSpec((1, ts, kcols), lambda bi, si: (bi, si, 0)),
            pl.BlockSpec((1, 1, 2 * e), lambda bi, si: (bi, 0, 0)),
            pl.BlockSpec((kcols, d), lambda bi, si: (0, 0)),
        ],
        out_specs=pl.BlockSpec((1, ts, d), lambda bi, si: (bi, si, 0)),
        out_shape=jax.ShapeDtypeStruct((b, s_total, d), jnp.float32),
        scratch_shapes=[pltpu.VMEM((kcols, d), jnp.bfloat16)],
    )(h, ls, b_aug)


# speedup vs baseline: 1.3853x; 1.1085x over previous
import functools

import jax
import jax.numpy as jnp
from jax import lax
from jax.experimental import pallas as pl
from jax.experimental.pallas import tpu as pltpu

_ALPHA = 16.0
_NEG = -1e30


def _moe_lora_body(x_ref, bias_ref, baug_ref, aaug_ref, out_ref,
                   bs_ref, h_ref, ls_ref, *, ts, nt, s_total, e, r, nb):
    bi = pl.program_id(0)
    si = pl.program_id(1)
    er = e * r
    parity = jax.lax.rem(bi, 2)

    @pl.when((bi >= 1) & (si == 0))
    def _router():
        logits = ls_ref[...] * (1.0 / s_total) + bias_ref[0]
        iota = lax.broadcasted_iota(jnp.int32, (1, e), 1)
        v0 = jnp.max(logits, keepdims=True)
        i0 = jnp.min(jnp.where(logits == v0, iota, e), keepdims=True)
        masked = jnp.where(iota == i0, _NEG, logits)
        v1 = jnp.max(masked, keepdims=True)
        i1 = jnp.min(jnp.where(masked == v1, iota, e), keepdims=True)
        t = jnp.exp(v1 - v0)
        scaling = _ALPHA / r
        wrow = (jnp.where(iota == i0, scaling / (1.0 + t), 0.0)
                + jnp.where(iota == i1, scaling * t / (1.0 + t), 0.0))
        rowexp = lax.broadcasted_iota(jnp.int32, (er + e, e), 0) // r
        lane = lax.broadcasted_iota(jnp.int32, (er + e, e), 1)
        wcol = jnp.sum(jnp.where(rowexp == lane, wrow, 0.0),
                       axis=1, keepdims=True)
        bs_ref[...] = (baug_ref[...] * wcol).astype(jnp.bfloat16)

    @pl.when(bi < nb)
    def _a_work():
        x = x_ref[0]
        hf = jax.lax.dot_general(
            x.astype(jnp.bfloat16), aaug_ref[...], (((1,), (1,)), ((), ())),
            preferred_element_type=jnp.float32)
        h_ref[parity, pl.ds(si * ts, ts), :] = hf.astype(jnp.bfloat16)
        part = jnp.sum(hf[:, er:er + e], axis=0, keepdims=True)

        @pl.when(si == 0)
        def _():
            ls_ref[...] = part

        @pl.when(si != 0)
        def _():
            ls_ref[...] = ls_ref[...] + part

    @pl.when(bi >= 1)
    def _b_work():
        h = h_ref[1 - parity, pl.ds(si * ts, ts), :]
        out_ref[0] = jax.lax.dot_general(
            h, bs_ref[...], (((1,), (0,)), ((), ())),
            preferred_element_type=jnp.float32)


def kernel(hidden_states, task_id, mode_id, W_gate, task_bias, mode_bias, A, Bw):
    b, s_total, d = hidden_states.shape
    e, r, _ = A.shape
    ts = 1024
    nt = s_total // ts

    a_aug = jnp.concatenate(
        [A.reshape(e * r, d), W_gate], axis=0).astype(jnp.bfloat16)
    b_aug = jnp.concatenate(
        [Bw.transpose(0, 2, 1).reshape(e * r, d),
         jnp.zeros((e, d), jnp.float32)], axis=0)
    bias = (jnp.take(task_bias, task_id, axis=0)
            + jnp.take(mode_bias, mode_id, axis=0))
    bias_row = bias.reshape(b, 1, e)

    body = functools.partial(_moe_lora_body, ts=ts, nt=nt,
                             s_total=s_total, e=e, r=r, nb=b)

    return pl.pallas_call(
        body,
        grid=(b + 1, nt),
        in_specs=[
            pl.BlockSpec((1, ts, d),
                         lambda bi, si, _b=b, _nt=nt: (
                             jnp.minimum(bi, _b - 1),
                             jnp.where(bi < _b, si, _nt - 1), 0)),
            pl.BlockSpec((1, 1, e),
                         lambda bi, si: (jnp.maximum(bi - 1, 0), 0, 0)),
            pl.BlockSpec((e * r + e, d), lambda bi, si: (0, 0)),
            pl.BlockSpec((e * r + e, d), lambda bi, si: (0, 0)),
        ],
        out_specs=pl.BlockSpec((1, ts, d),
                               lambda bi, si: (jnp.maximum(bi - 1, 0),
                                               jnp.where(bi >= 1, si, 0), 0)),
        out_shape=jax.ShapeDtypeStruct((b, s_total, d), jnp.float32),
        scratch_shapes=[
            pltpu.VMEM((e * r + e, d), jnp.bfloat16),
            pltpu.VMEM((2, s_total, e * r + e), jnp.bfloat16),
            pltpu.VMEM((1, e), jnp.float32),
        ],
    )(hidden_states, bias_row, b_aug, a_aug)
